# Initial kernel scaffold; baseline (speedup 1.0000x reference)
import jax
import jax.numpy as jnp
from jax.experimental import pallas as pl


def kernel(x, edge_index, W1, b1, W2, b2):
    raise NotImplementedError("write your pallas kernel here")


# speedup vs baseline: 7.1946x; 7.1946x over previous
import functools

import jax
import jax.numpy as jnp
from jax import lax
from jax.experimental import pallas as pl
from jax.experimental.pallas import tpu as pltpu
from jax.experimental.pallas import tpu_sc as plsc

_NC = 2
_NS = 16
_CH = 128
_ZR = 64
_BM = 256
_DEGW = 16


def _round_up(v, m):
    return (v + m - 1) // m * m


def _make_sc_scatter(NP, D, EP, gather):
    n_idx_rows = EP // _CH
    rows_per_worker = n_idx_rows // (_NC * _NS)
    acc_rows_per_sub = NP // _NS
    mesh = plsc.VectorSubcoreMesh(core_axis_name="c", subcore_axis_name="s")

    scratch = [
        pltpu.VMEM((rows_per_worker, _CH), jnp.int32),
        pltpu.VMEM((_ZR, D), jnp.float32),
        pltpu.VMEM((_CH, D), jnp.float32),
        pltpu.VMEM_SHARED((NP, D), jnp.float32),
        pltpu.SemaphoreType.DMA,
    ]
    if gather:
        scratch.insert(0, pltpu.VMEM((rows_per_worker, _CH), jnp.int32))

    @functools.partial(
        pl.kernel,
        mesh=mesh,
        out_type=jax.ShapeDtypeStruct((_NC * NP, D), jnp.float32),
        scratch_types=scratch,
        compiler_params=pltpu.CompilerParams(use_tc_tiling_on_sc=False),
    )
    def k(*refs):
        if gather:
            g_hbm, src_hbm, dst_hbm, out_hbm, src_v, dst_v, zbuf, rbuf, acc, sem = refs
        else:
            dst_hbm, out_hbm, dst_v, zbuf, rbuf, acc, sem = refs
        c = lax.axis_index("c")
        s = lax.axis_index("s")

        def zstore(i, _):
            r = i // (D // 16)
            col = (i % (D // 16)) * 16
            zbuf[r, pl.ds(col, 16)] = jnp.zeros((16,), jnp.float32)
            return 0
        lax.fori_loop(0, _ZR * D // 16, zstore, 0)
        if not gather:
            def ostore(i, _):
                r = i // (D // 16)
                col = (i % (D // 16)) * 16
                rbuf[r, pl.ds(col, 16)] = jnp.ones((16,), jnp.float32)
                return 0
            lax.fori_loop(0, _CH * D // 16, ostore, 0)

        row_base = s * acc_rows_per_sub
        def zcopy(j, _):
            pltpu.sync_copy(zbuf, acc.at[pl.ds(row_base + j * _ZR, _ZR)])
            return 0
        lax.fori_loop(0, acc_rows_per_sub // _ZR, zcopy, 0)
        plsc.subcore_barrier()

        wrow = (c * _NS + s) * rows_per_worker
        pltpu.sync_copy(dst_hbm.at[pl.ds(wrow, rows_per_worker)], dst_v)
        if gather:
            pltpu.sync_copy(src_hbm.at[pl.ds(wrow, rows_per_worker)], src_v)

        def step(j, _):
            if gather:
                pltpu.async_copy(g_hbm.at[src_v.at[j]], rbuf, sem).wait()
            pltpu.sync_copy(rbuf, acc.at[dst_v.at[j]], add=True)
            return 0
        lax.fori_loop(0, rows_per_worker, step, 0)
        plsc.subcore_barrier()

        pltpu.sync_copy(acc.at[pl.ds(row_base, acc_rows_per_sub)],
                        out_hbm.at[pl.ds(c * NP + row_base, acc_rows_per_sub)])

    return k


def _dis(dega_ref, degb_ref):
    deg = dega_ref[...][:, :1] + degb_ref[...][:, :1] + 1.0
    return lax.rsqrt(deg)


def _mm1_body(dega_ref, degb_ref, x_ref, w_ref, o_ref):
    dis = _dis(dega_ref, degb_ref)
    o_ref[...] = jnp.dot(x_ref[...], w_ref[...],
                         preferred_element_type=jnp.float32) * dis


def _mid_body(dega_ref, degb_ref, s1a_ref, s1b_ref, g1_ref, b1_ref, w2_ref, o_ref):
    dis = _dis(dega_ref, degb_ref)
    u = jnp.maximum(dis * (s1a_ref[...] + s1b_ref[...] + g1_ref[...])
                    + b1_ref[...], 0.0)
    o_ref[...] = jnp.dot(u, w2_ref[...], preferred_element_type=jnp.float32) * dis


def _fin_body(dega_ref, degb_ref, s2a_ref, s2b_ref, g2_ref, b2_ref, o_ref):
    dis = _dis(dega_ref, degb_ref)
    o_ref[...] = dis * (s2a_ref[...] + s2b_ref[...] + g2_ref[...]) + b2_ref[...]


def _row_spec(d):
    return pl.BlockSpec((_BM, d), lambda i: (i, 0))


def _full_spec(r, c):
    return pl.BlockSpec((r, c), lambda i: (0, 0))


def kernel(x, edge_index, W1, b1, W2, b2):
    N, F = x.shape
    E = edge_index.shape[1]
    D1 = W1.shape[1]
    D2 = W2.shape[1]
    NP = _round_up(N + 1, _NS * _ZR)
    EP = _round_up(E, _NC * _NS * _CH)
    FP = _round_up(F, 128)

    pad = jnp.full((EP - E,), N, jnp.int32)
    src = jnp.concatenate([edge_index[0], pad]).reshape(EP // _CH, _CH)
    dst = jnp.concatenate([edge_index[1], pad]).reshape(EP // _CH, _CH)
    xp = jnp.zeros((NP, FP), jnp.float32).at[:N, :F].set(x)
    w1p = jnp.zeros((FP, D1), jnp.float32).at[:F].set(W1)

    degp = _make_sc_scatter(NP, _DEGW, EP, gather=False)(dst)
    dega, degb = degp[:NP], degp[NP:]

    grid = (NP // _BM,)
    deg_spec = _row_spec(_DEGW)

    g1 = pl.pallas_call(
        _mm1_body,
        grid=grid,
        in_specs=[deg_spec, deg_spec, _row_spec(FP), _full_spec(FP, D1)],
        out_specs=_row_spec(D1),
        out_shape=jax.ShapeDtypeStruct((NP, D1), jnp.float32),
    )(dega, degb, xp, w1p)

    s1 = _make_sc_scatter(NP, D1, EP, gather=True)(g1, src, dst)

    g2 = pl.pallas_call(
        _mid_body,
        grid=grid,
        in_specs=[deg_spec, deg_spec, _row_spec(D1), _row_spec(D1), _row_spec(D1),
                  _full_spec(1, D1), _full_spec(D1, D2)],
        out_specs=_row_spec(D2),
        out_shape=jax.ShapeDtypeStruct((NP, D2), jnp.float32),
    )(dega, degb, s1[:NP], s1[NP:], g1, b1.reshape(1, D1), W2)

    s2 = _make_sc_scatter(NP, D2, EP, gather=True)(g2, src, dst)

    out = pl.pallas_call(
        _fin_body,
        grid=grid,
        in_specs=[deg_spec, deg_spec, _row_spec(D2), _row_spec(D2), _row_spec(D2),
                  _full_spec(1, D2)],
        out_specs=_row_spec(D2),
        out_shape=jax.ShapeDtypeStruct((NP, D2), jnp.float32),
    )(dega, degb, s2[:NP], s2[NP:], g2, b2.reshape(1, D2))

    return out[:N]


# speedup vs baseline: 8.7937x; 1.2223x over previous
import functools

import jax
import jax.numpy as jnp
from jax import lax
from jax.experimental import pallas as pl
from jax.experimental.pallas import tpu as pltpu
from jax.experimental.pallas import tpu_sc as plsc

_NC = 2
_NS = 16
_CH = 96
_ZR = 64
_BM = 256
_DEGW = 16


def _round_up(v, m):
    return (v + m - 1) // m * m


def _make_sc_scatter(NP, D, EP, gather):
    n_idx_rows = EP // _CH
    rows_per_worker = n_idx_rows // (_NC * _NS)
    acc_rows_per_sub = NP // _NS
    mesh = plsc.VectorSubcoreMesh(core_axis_name="c", subcore_axis_name="s")

    scratch = [
        pltpu.VMEM((rows_per_worker, _CH), jnp.int32),
        pltpu.VMEM((_ZR, D), jnp.float32),
        pltpu.VMEM((_CH, D), jnp.float32),
        pltpu.VMEM_SHARED((NP, D), jnp.float32),
        pltpu.SemaphoreType.DMA,
    ]
    if gather:
        scratch.insert(0, pltpu.VMEM((rows_per_worker, _CH), jnp.int32))
        scratch += [pltpu.VMEM((_CH, D), jnp.float32),
                    pltpu.SemaphoreType.DMA]

    @functools.partial(
        pl.kernel,
        mesh=mesh,
        out_type=jax.ShapeDtypeStruct((_NC * NP, D), jnp.float32),
        scratch_types=scratch,
        compiler_params=pltpu.CompilerParams(use_tc_tiling_on_sc=False),
    )
    def k(*refs):
        if gather:
            (g_hbm, src_hbm, dst_hbm, out_hbm, src_v, dst_v, zbuf, rbuf, acc,
             sem, rbuf1, sem1) = refs
        else:
            dst_hbm, out_hbm, dst_v, zbuf, rbuf, acc, sem = refs
        c = lax.axis_index("c")
        s = lax.axis_index("s")

        def zstore(i, _):
            r = i // (D // 16)
            col = (i % (D // 16)) * 16
            zbuf[r, pl.ds(col, 16)] = jnp.zeros((16,), jnp.float32)
            return 0
        lax.fori_loop(0, _ZR * D // 16, zstore, 0)
        if not gather:
            def ostore(i, _):
                r = i // (D // 16)
                col = (i % (D // 16)) * 16
                rbuf[r, pl.ds(col, 16)] = jnp.ones((16,), jnp.float32)
                return 0
            lax.fori_loop(0, _CH * D // 16, ostore, 0)

        row_base = s * acc_rows_per_sub
        def zcopy(j, _):
            pltpu.sync_copy(zbuf, acc.at[pl.ds(row_base + j * _ZR, _ZR)])
            return 0
        lax.fori_loop(0, acc_rows_per_sub // _ZR, zcopy, 0)
        plsc.subcore_barrier()

        wrow = (c * _NS + s) * rows_per_worker
        pltpu.sync_copy(dst_hbm.at[pl.ds(wrow, rows_per_worker)], dst_v)
        if gather:
            pltpu.sync_copy(src_hbm.at[pl.ds(wrow, rows_per_worker)], src_v)

        if gather:
            def _g(j, buf, s):
                return pltpu.make_async_copy(g_hbm.at[src_v.at[j]], buf, s)
            _g(0, rbuf, sem).start()
            if rows_per_worker > 1:
                _g(1, rbuf1, sem1).start()

            def step(j2, _):
                j = 2 * j2
                _g(j, rbuf, sem).wait()
                pltpu.sync_copy(rbuf, acc.at[dst_v.at[j]], add=True)
                @pl.when(j + 2 < rows_per_worker)
                def _():
                    _g(j + 2, rbuf, sem).start()
                @pl.when(j + 1 < rows_per_worker)
                def _():
                    _g(j + 1, rbuf1, sem1).wait()
                    pltpu.sync_copy(rbuf1, acc.at[dst_v.at[j + 1]], add=True)
                    @pl.when(j + 3 < rows_per_worker)
                    def _():
                        _g(j + 3, rbuf1, sem1).start()
                return 0
            lax.fori_loop(0, (rows_per_worker + 1) // 2, step, 0)
        else:
            def step(j, _):
                pltpu.sync_copy(rbuf, acc.at[dst_v.at[j]], add=True)
                return 0
            lax.fori_loop(0, rows_per_worker, step, 0)
        plsc.subcore_barrier()

        pltpu.sync_copy(acc.at[pl.ds(row_base, acc_rows_per_sub)],
                        out_hbm.at[pl.ds(c * NP + row_base, acc_rows_per_sub)])

    return k


def _dis(dega_ref, degb_ref):
    deg = dega_ref[...][:, :1] + degb_ref[...][:, :1] + 1.0
    return lax.rsqrt(deg)


def _mm1_body(dega_ref, degb_ref, x_ref, w_ref, o_ref):
    dis = _dis(dega_ref, degb_ref)
    o_ref[...] = jnp.dot(x_ref[...], w_ref[...],
                         preferred_element_type=jnp.float32) * dis


def _mid_body(dega_ref, degb_ref, s1a_ref, s1b_ref, g1_ref, b1_ref, w2_ref, o_ref):
    dis = _dis(dega_ref, degb_ref)
    u = jnp.maximum(dis * (s1a_ref[...] + s1b_ref[...] + g1_ref[...])
                    + b1_ref[...], 0.0)
    o_ref[...] = jnp.dot(u, w2_ref[...], preferred_element_type=jnp.float32) * dis


def _fin_body(dega_ref, degb_ref, s2a_ref, s2b_ref, g2_ref, b2_ref, o_ref):
    dis = _dis(dega_ref, degb_ref)
    o_ref[...] = dis * (s2a_ref[...] + s2b_ref[...] + g2_ref[...]) + b2_ref[...]


def _row_spec(d):
    return pl.BlockSpec((_BM, d), lambda i: (i, 0))


def _full_spec(r, c):
    return pl.BlockSpec((r, c), lambda i: (0, 0))


def kernel(x, edge_index, W1, b1, W2, b2):
    N, F = x.shape
    E = edge_index.shape[1]
    D1 = W1.shape[1]
    D2 = W2.shape[1]
    NP = _round_up(N + 1, _NS * _ZR)
    EP = _round_up(E, _NC * _NS * _CH)
    FP = _round_up(F, 128)

    pad = jnp.full((EP - E,), N, jnp.int32)
    src = jnp.concatenate([edge_index[0], pad]).reshape(EP // _CH, _CH)
    dst = jnp.concatenate([edge_index[1], pad]).reshape(EP // _CH, _CH)
    xp = jnp.zeros((NP, FP), jnp.float32).at[:N, :F].set(x)
    w1p = jnp.zeros((FP, D1), jnp.float32).at[:F].set(W1)

    degp = _make_sc_scatter(NP, _DEGW, EP, gather=False)(dst)
    dega, degb = degp[:NP], degp[NP:]

    grid = (NP // _BM,)
    deg_spec = _row_spec(_DEGW)

    g1 = pl.pallas_call(
        _mm1_body,
        grid=grid,
        in_specs=[deg_spec, deg_spec, _row_spec(FP), _full_spec(FP, D1)],
        out_specs=_row_spec(D1),
        out_shape=jax.ShapeDtypeStruct((NP, D1), jnp.float32),
    )(dega, degb, xp, w1p)

    s1 = _make_sc_scatter(NP, D1, EP, gather=True)(g1, src, dst)

    g2 = pl.pallas_call(
        _mid_body,
        grid=grid,
        in_specs=[deg_spec, deg_spec, _row_spec(D1), _row_spec(D1), _row_spec(D1),
                  _full_spec(1, D1), _full_spec(D1, D2)],
        out_specs=_row_spec(D2),
        out_shape=jax.ShapeDtypeStruct((NP, D2), jnp.float32),
    )(dega, degb, s1[:NP], s1[NP:], g1, b1.reshape(1, D1), W2)

    s2 = _make_sc_scatter(NP, D2, EP, gather=True)(g2, src, dst)

    out = pl.pallas_call(
        _fin_body,
        grid=grid,
        in_specs=[deg_spec, deg_spec, _row_spec(D2), _row_spec(D2), _row_spec(D2),
                  _full_spec(1, D2)],
        out_specs=_row_spec(D2),
        out_shape=jax.ShapeDtypeStruct((NP, D2), jnp.float32),
    )(dega, degb, s2[:NP], s2[NP:], g2, b2.reshape(1, D2))

    return out[:N]


# speedup vs baseline: 10.4054x; 1.1833x over previous
import functools

import jax
import jax.numpy as jnp
from jax import lax
from jax.experimental import pallas as pl
from jax.experimental.pallas import tpu as pltpu
from jax.experimental.pallas import tpu_sc as plsc

_NC = 2
_NS = 16
_CH = 96
_ZR = 64
_BM = 256
_DEGW = 16


def _round_up(v, m):
    return (v + m - 1) // m * m


def _make_sc_scatter(NP, D, EP, gather):
    n_idx_rows = EP // _CH
    rows_per_worker = n_idx_rows // (_NC * _NS)
    acc_rows_per_sub = NP // _NS
    mesh = plsc.VectorSubcoreMesh(core_axis_name="c", subcore_axis_name="s")

    scratch = [
        pltpu.VMEM((rows_per_worker, _CH), jnp.int32),
        pltpu.VMEM((_ZR, D), jnp.float32),
        pltpu.VMEM((_CH, D), jnp.float32),
        pltpu.VMEM_SHARED((NP, D), jnp.float32),
        pltpu.SemaphoreType.DMA,
    ]
    if gather:
        scratch.insert(0, pltpu.VMEM((rows_per_worker, _CH), jnp.int32))
        scratch += [pltpu.VMEM((_CH, D), jnp.float32),
                    pltpu.SemaphoreType.DMA]

    @functools.partial(
        pl.kernel,
        mesh=mesh,
        out_type=jax.ShapeDtypeStruct((_NC * NP, D), jnp.float32),
        scratch_types=scratch,
        compiler_params=pltpu.CompilerParams(use_tc_tiling_on_sc=False),
    )
    def k(*refs):
        if gather:
            (g_hbm, src_hbm, dst_hbm, out_hbm, src_v, dst_v, zbuf, rbuf, acc,
             sem, rbuf1, sem1) = refs
        else:
            dst_hbm, out_hbm, dst_v, zbuf, rbuf, acc, sem = refs
        c = lax.axis_index("c")
        s = lax.axis_index("s")

        def zstore(i, _):
            r = i // (D // 16)
            col = (i % (D // 16)) * 16
            zbuf[r, pl.ds(col, 16)] = jnp.zeros((16,), jnp.float32)
            return 0
        lax.fori_loop(0, _ZR * D // 16, zstore, 0)
        if not gather:
            def ostore(i, _):
                r = i // (D // 16)
                col = (i % (D // 16)) * 16
                rbuf[r, pl.ds(col, 16)] = jnp.ones((16,), jnp.float32)
                return 0
            lax.fori_loop(0, _CH * D // 16, ostore, 0)

        row_base = s * acc_rows_per_sub
        def zcopy(j, _):
            pltpu.sync_copy(zbuf, acc.at[pl.ds(row_base + j * _ZR, _ZR)])
            return 0
        lax.fori_loop(0, acc_rows_per_sub // _ZR, zcopy, 0)
        plsc.subcore_barrier()

        wrow = (c * _NS + s) * rows_per_worker
        pltpu.sync_copy(dst_hbm.at[pl.ds(wrow, rows_per_worker)], dst_v)
        if gather:
            pltpu.sync_copy(src_hbm.at[pl.ds(wrow, rows_per_worker)], src_v)

        if gather:
            def _g(j, buf, s):
                return pltpu.make_async_copy(g_hbm.at[src_v.at[j]], buf, s)
            _g(0, rbuf, sem).start()
            if rows_per_worker > 1:
                _g(1, rbuf1, sem1).start()

            def step(j2, _):
                j = 2 * j2
                _g(j, rbuf, sem).wait()
                pltpu.sync_copy(rbuf, acc.at[dst_v.at[j]], add=True)
                @pl.when(j + 2 < rows_per_worker)
                def _():
                    _g(j + 2, rbuf, sem).start()
                @pl.when(j + 1 < rows_per_worker)
                def _():
                    _g(j + 1, rbuf1, sem1).wait()
                    pltpu.sync_copy(rbuf1, acc.at[dst_v.at[j + 1]], add=True)
                    @pl.when(j + 3 < rows_per_worker)
                    def _():
                        _g(j + 3, rbuf1, sem1).start()
                return 0
            lax.fori_loop(0, (rows_per_worker + 1) // 2, step, 0)
        else:
            def step(j, _):
                pltpu.sync_copy(rbuf, acc.at[dst_v.at[j]], add=True)
                return 0
            lax.fori_loop(0, rows_per_worker, step, 0)
        plsc.subcore_barrier()

        pltpu.sync_copy(acc.at[pl.ds(row_base, acc_rows_per_sub)],
                        out_hbm.at[pl.ds(c * NP + row_base, acc_rows_per_sub)])

    return k


def _dis(dega_ref, degb_ref):
    deg = dega_ref[...][:, :1] + degb_ref[...][:, :1] + 1.0
    return lax.rsqrt(deg)


def _half_specs(d, nblk):
    return [pl.BlockSpec((_BM, d), lambda i: (i, 0)),
            pl.BlockSpec((_BM, d), lambda i, nb=nblk: (i + nb, 0))]


def _mm1_body(dega_ref, degb_ref, x_ref, w_ref, o_ref):
    dis = _dis(dega_ref, degb_ref)
    o_ref[...] = jnp.dot(x_ref[...], w_ref[...],
                         preferred_element_type=jnp.float32) * dis


def _mid_body(dega_ref, degb_ref, s1a_ref, s1b_ref, g1_ref, b1_ref, w2_ref, o_ref):
    dis = _dis(dega_ref, degb_ref)
    u = jnp.maximum(dis * (s1a_ref[...] + s1b_ref[...] + g1_ref[...])
                    + b1_ref[...], 0.0)
    o_ref[...] = jnp.dot(u, w2_ref[...], preferred_element_type=jnp.float32) * dis


def _fin_body(dega_ref, degb_ref, s2a_ref, s2b_ref, g2_ref, b2_ref, o_ref):
    dis = _dis(dega_ref, degb_ref)
    o_ref[...] = dis * (s2a_ref[...] + s2b_ref[...] + g2_ref[...]) + b2_ref[...]


def _row_spec(d):
    return pl.BlockSpec((_BM, d), lambda i: (i, 0))


def _full_spec(r, c):
    return pl.BlockSpec((r, c), lambda i: (0, 0))


def kernel(x, edge_index, W1, b1, W2, b2):
    N, F = x.shape
    E = edge_index.shape[1]
    D1 = W1.shape[1]
    D2 = W2.shape[1]
    NP = _round_up(N + 1, _NS * _ZR)
    EP = _round_up(E, _NC * _NS * _CH)

    pad = jnp.full((EP - E,), N, jnp.int32)
    src = jnp.concatenate([edge_index[0], pad]).reshape(EP // _CH, _CH)
    dst = jnp.concatenate([edge_index[1], pad]).reshape(EP // _CH, _CH)

    degp = _make_sc_scatter(NP, _DEGW, EP, gather=False)(dst)

    nblk = NP // _BM
    grid = (nblk,)
    deg_specs = _half_specs(_DEGW, nblk)

    g1 = pl.pallas_call(
        _mm1_body,
        grid=grid,
        in_specs=deg_specs + [_row_spec(F), _full_spec(F, D1)],
        out_specs=_row_spec(D1),
        out_shape=jax.ShapeDtypeStruct((NP, D1), jnp.float32),
    )(degp, degp, x, W1)

    s1 = _make_sc_scatter(NP, D1, EP, gather=True)(g1, src, dst)

    g2 = pl.pallas_call(
        _mid_body,
        grid=grid,
        in_specs=deg_specs + _half_specs(D1, nblk) + [_row_spec(D1),
                  _full_spec(1, D1), _full_spec(D1, D2)],
        out_specs=_row_spec(D2),
        out_shape=jax.ShapeDtypeStruct((NP, D2), jnp.float32),
    )(degp, degp, s1, s1, g1, b1.reshape(1, D1), W2)

    s2 = _make_sc_scatter(NP, D2, EP, gather=True)(g2, src, dst)

    out = pl.pallas_call(
        _fin_body,
        grid=grid,
        in_specs=deg_specs + _half_specs(D2, nblk) + [_row_spec(D2),
                  _full_spec(1, D2)],
        out_specs=_row_spec(D2),
        out_shape=jax.ShapeDtypeStruct((N, D2), jnp.float32),
    )(degp, degp, s2, s2, g2, b2.reshape(1, D2))

    return out
